# baseline (speedup 1.0000x reference)
import functools

import jax
import jax.numpy as jnp
from jax import lax
from jax.experimental import pallas as pl
from jax.experimental.pallas import tpu as pltpu
from jax.experimental.pallas import tpu_sc as plsc

BATCH = 4096
HIST = 200
EMBED_DIM = 64

NC = 2
NS = 16
NW = NC * NS

B = BATCH * HIST
B_PER_W = B // NW
CHUNK = 256
N_CHUNKS = B_PER_W // CHUNK

DEPTH = 4
LOOK = 2


def _gather_body(idx_hbm, table_hbm, out_hbm, idx_v, rows_v, gsem, ssem):
    wid = lax.axis_index("s") * NC + lax.axis_index("c")
    base = wid * B_PER_W
    pltpu.sync_copy(idx_hbm.at[pl.ds(wid * N_CHUNKS, N_CHUNKS)], idx_v)

    def start_gather(c, b):
        pltpu.async_copy(table_hbm.at[idx_v.at[c]], rows_v.at[b], gsem.at[b])

    def wait_gather(c, b):
        pltpu.make_async_copy(table_hbm.at[idx_v.at[c]], rows_v.at[b],
                              gsem.at[b]).wait()

    def start_store(c, b):
        pltpu.async_copy(rows_v.at[b], out_hbm.at[pl.ds(base + c * CHUNK, CHUNK)],
                         ssem.at[b])

    def wait_store(c, b):
        pltpu.make_async_copy(rows_v.at[b],
                              out_hbm.at[pl.ds(base + c * CHUNK, CHUNK)],
                              ssem.at[b]).wait()

    for c0 in range(LOOK):
        start_gather(c0, c0 % DEPTH)

    def body(s, _):
        for b in range(DEPTH):
            c = s * DEPTH + b
            bl = (b + LOOK) % DEPTH

            @pl.when(c + LOOK < N_CHUNKS)
            def _():
                @pl.when(c + LOOK >= DEPTH)
                def _():
                    wait_store(c + LOOK - DEPTH, bl)
                start_gather(c + LOOK, bl)

            wait_gather(c, b)
            start_store(c, b)
        return 0

    lax.fori_loop(0, N_CHUNKS // DEPTH, body, 0)

    for c in range(N_CHUNKS - (DEPTH - LOOK), N_CHUNKS):
        wait_store(c, c % DEPTH)


@jax.jit
def _run(idx2d, table):
    mesh = plsc.VectorSubcoreMesh(core_axis_name="c", subcore_axis_name="s")
    kfn = pl.kernel(
        _gather_body,
        mesh=mesh,
        compiler_params=pltpu.CompilerParams(use_tc_tiling_on_sc=False),
        out_type=jax.ShapeDtypeStruct((B, EMBED_DIM), jnp.float32),
        scratch_types=[
            pltpu.VMEM((N_CHUNKS, CHUNK), jnp.int32),
            pltpu.VMEM((DEPTH, CHUNK, EMBED_DIM), jnp.float32),
            pltpu.SemaphoreType.DMA((DEPTH,)),
            pltpu.SemaphoreType.DMA((DEPTH,)),
        ],
    )
    return kfn(idx2d, table)


def kernel(word_index, table):
    idx2d = word_index.astype(jnp.int32).reshape(NW * N_CHUNKS, CHUNK)
    out = _run(idx2d, table)
    return out.reshape(BATCH, HIST, EMBED_DIM)


# speedup vs baseline: 1.0011x; 1.0011x over previous
import functools

import jax
import jax.numpy as jnp
from jax import lax
from jax.experimental import pallas as pl
from jax.experimental.pallas import tpu as pltpu
from jax.experimental.pallas import tpu_sc as plsc

BATCH = 4096
HIST = 200
EMBED_DIM = 64

NC = 2
NS = 16
NW = NC * NS

B = BATCH * HIST
B_PER_W = B // NW
CHUNK = 256
N_CHUNKS = B_PER_W // CHUNK

DEPTH = 5
LOOK = 3


def _gather_body(idx_hbm, table_hbm, out_hbm, idx_v, rows_v, gsem, ssem):
    wid = lax.axis_index("s") * NC + lax.axis_index("c")
    base = wid * B_PER_W
    pltpu.sync_copy(idx_hbm.at[pl.ds(wid * N_CHUNKS, N_CHUNKS)], idx_v)

    def start_gather(c, b):
        pltpu.async_copy(table_hbm.at[idx_v.at[c]], rows_v.at[b], gsem.at[b])

    def wait_gather(c, b):
        pltpu.make_async_copy(table_hbm.at[idx_v.at[c]], rows_v.at[b],
                              gsem.at[b]).wait()

    def start_store(c, b):
        pltpu.async_copy(rows_v.at[b], out_hbm.at[pl.ds(base + c * CHUNK, CHUNK)],
                         ssem.at[b])

    def wait_store(c, b):
        pltpu.make_async_copy(rows_v.at[b],
                              out_hbm.at[pl.ds(base + c * CHUNK, CHUNK)],
                              ssem.at[b]).wait()

    for c0 in range(LOOK):
        start_gather(c0, c0 % DEPTH)

    def body(s, _):
        for b in range(DEPTH):
            c = s * DEPTH + b
            bl = (b + LOOK) % DEPTH

            @pl.when(c + LOOK < N_CHUNKS)
            def _():
                @pl.when(c + LOOK >= DEPTH)
                def _():
                    wait_store(c + LOOK - DEPTH, bl)
                start_gather(c + LOOK, bl)

            wait_gather(c, b)
            start_store(c, b)
        return 0

    lax.fori_loop(0, N_CHUNKS // DEPTH, body, 0)

    for c in range(N_CHUNKS - (DEPTH - LOOK), N_CHUNKS):
        wait_store(c, c % DEPTH)


@jax.jit
def _run(idx2d, table):
    mesh = plsc.VectorSubcoreMesh(core_axis_name="c", subcore_axis_name="s")
    kfn = pl.kernel(
        _gather_body,
        mesh=mesh,
        compiler_params=pltpu.CompilerParams(use_tc_tiling_on_sc=False),
        out_type=jax.ShapeDtypeStruct((B, EMBED_DIM), jnp.float32),
        scratch_types=[
            pltpu.VMEM((N_CHUNKS, CHUNK), jnp.int32),
            pltpu.VMEM((DEPTH, CHUNK, EMBED_DIM), jnp.float32),
            pltpu.SemaphoreType.DMA((DEPTH,)),
            pltpu.SemaphoreType.DMA((DEPTH,)),
        ],
    )
    return kfn(idx2d, table)


def kernel(word_index, table):
    idx2d = word_index.astype(jnp.int32).reshape(NW * N_CHUNKS, CHUNK)
    out = _run(idx2d, table)
    return out.reshape(BATCH, HIST, EMBED_DIM)
